# Initial kernel scaffold; baseline (speedup 1.0000x reference)
import jax
import jax.numpy as jnp
from jax.experimental import pallas as pl


def kernel(s_embed, r_embed, e_embed, senders, receivers, edge_contr, norm, W_s, b_s, W_r, b_r, W_e, W_out, scale1, scale2):
    raise NotImplementedError("write your pallas kernel here")


# speedup vs baseline: 2.3985x; 2.3985x over previous
import functools

import jax
import jax.numpy as jnp
import numpy as np
from jax import lax
from jax.experimental import pallas as pl
from jax.experimental.pallas import tpu as pltpu
from jax.experimental.pallas import tpu_sc as plsc

GAIN = 1.6765512
N = 10000
D = 128
MSG = 128
OUT = 128

N_PAD = 10240
NUM_WORKERS = 32
CHUNK = 64
ROW_BLK = 400
EW_BLK = 2048


def _silu_gain(z):
  return z / (1.0 + jnp.exp(-z)) * GAIN


def _node_mm_body(xs, ws, bs, xr, wr, br, s_out, r_out):
  s_out[...] = jnp.dot(xs[...], ws[...], preferred_element_type=jnp.float32) + bs[...]
  r_out[...] = jnp.dot(xr[...], wr[...], preferred_element_type=jnp.float32) + br[...]


def _node_mm(s_embed, ws, bs, r_embed, wr, br):
  n = s_embed.shape[0]
  grid = n // ROW_BLK
  blk = lambda i: (i, 0)
  fixed = lambda i: (0, 0)
  return pl.pallas_call(
      _node_mm_body,
      grid=(grid,),
      in_specs=[
          pl.BlockSpec((ROW_BLK, D), blk),
          pl.BlockSpec((D, MSG), fixed),
          pl.BlockSpec((1, MSG), fixed),
          pl.BlockSpec((ROW_BLK, D), blk),
          pl.BlockSpec((D, MSG), fixed),
          pl.BlockSpec((1, MSG), fixed),
      ],
      out_specs=[pl.BlockSpec((ROW_BLK, MSG), blk)] * 2,
      out_shape=[jax.ShapeDtypeStruct((n, MSG), jnp.float32)] * 2,
  )(s_embed, ws, bs, r_embed, wr, br)


def _ew_body(ee, we, out):
  out[...] = jnp.dot(ee[...], we[...], preferred_element_type=jnp.float32)


def _ew_mm(e_pad, we):
  e_rows, de = e_pad.shape
  grid = e_rows // EW_BLK
  return pl.pallas_call(
      _ew_body,
      grid=(grid,),
      in_specs=[
          pl.BlockSpec((EW_BLK, de), lambda i: (i, 0)),
          pl.BlockSpec((de, MSG), lambda i: (0, 0)),
      ],
      out_specs=pl.BlockSpec((EW_BLK, MSG), lambda i: (i, 0)),
      out_shape=jax.ShapeDtypeStruct((e_rows, MSG), jnp.float32),
  )(e_pad, we)


def _make_sc_edge(e_pad_rows):
  epw = e_pad_rows // NUM_WORKERS
  chunks = epw // CHUNK
  rows_per_tile = N_PAD // 16

  mesh = plsc.VectorSubcoreMesh(core_axis_name="c", subcore_axis_name="s")

  @functools.partial(
      pl.kernel,
      mesh=mesh,
      out_type=(
          jax.ShapeDtypeStruct((N_PAD, MSG), jnp.float32),
          jax.ShapeDtypeStruct((N_PAD, MSG), jnp.float32),
      ),
      scratch_types=[
          pltpu.VMEM((CHUNK,), jnp.int32),
          pltpu.VMEM((CHUNK,), jnp.int32),
          pltpu.VMEM((CHUNK, MSG), jnp.float32),
          pltpu.VMEM((CHUNK, MSG), jnp.float32),
          pltpu.VMEM((CHUNK, MSG), jnp.float32),
          pltpu.VMEM_SHARED((N_PAD, MSG), jnp.float32),
          pltpu.SemaphoreType.DMA,
          pltpu.SemaphoreType.DMA,
      ],
  )
  def sc_edge(s_hbm, r_hbm, ew_hbm, send_hbm, recv_hbm, zeros_hbm,
              out0, out1,
              idx_s, idx_r, s_rows, r_rows, ew_rows, msg_acc,
              sem_s, sem_r):
    c = lax.axis_index("c")
    s = lax.axis_index("s")
    wid = s * 2 + c
    tile_rows = pl.ds(s * rows_per_tile, rows_per_tile)
    pltpu.sync_copy(zeros_hbm.at[tile_rows], msg_acc.at[tile_rows])
    plsc.subcore_barrier()

    base_w = wid * epw

    def chunk_body(k, carry):
      base = base_w + k * CHUNK
      pltpu.sync_copy(send_hbm.at[pl.ds(base, CHUNK)], idx_s)
      pltpu.sync_copy(recv_hbm.at[pl.ds(base, CHUNK)], idx_r)
      g_s = pltpu.async_copy(s_hbm.at[idx_s], s_rows, sem_s)
      g_r = pltpu.async_copy(r_hbm.at[idx_r], r_rows, sem_r)
      pltpu.sync_copy(ew_hbm.at[pl.ds(base, CHUNK)], ew_rows)
      g_s.wait()
      g_r.wait()

      def edge_body(i, carry2):
        for j in range(MSG // 16):
          sl = pl.ds(j * 16, 16)
          x = s_rows[i, sl] + r_rows[i, sl]
          y = x / (1.0 + jnp.exp(-x))
          s_rows[i, sl] = y * ew_rows[i, sl]
        return carry2

      lax.fori_loop(0, CHUNK, edge_body, 0)
      pltpu.sync_copy(s_rows, msg_acc.at[idx_r], add=True)
      return carry

    lax.fori_loop(0, chunks, chunk_body, 0)
    plsc.subcore_barrier()

    @pl.when(c == 0)
    def _():
      pltpu.sync_copy(msg_acc.at[tile_rows], out0.at[tile_rows])

    @pl.when(c == 1)
    def _():
      pltpu.sync_copy(msg_acc.at[tile_rows], out1.at[tile_rows])

  return sc_edge


def _out_body(p0, p1, nrm, w, out):
  msg = (p0[...] + p1[...]) * nrm[...]
  z = jnp.dot(msg, w[...], preferred_element_type=jnp.float32)
  out[...] = _silu_gain(z)


def _out_mm(p0, p1, norm2, w_out):
  grid = N // ROW_BLK
  blk = lambda i: (i, 0)
  fixed = lambda i: (0, 0)
  return pl.pallas_call(
      _out_body,
      grid=(grid,),
      in_specs=[
          pl.BlockSpec((ROW_BLK, MSG), blk),
          pl.BlockSpec((ROW_BLK, MSG), blk),
          pl.BlockSpec((ROW_BLK, 1), blk),
          pl.BlockSpec((MSG, OUT), fixed),
      ],
      out_specs=pl.BlockSpec((ROW_BLK, OUT), blk),
      out_shape=jax.ShapeDtypeStruct((N, OUT), jnp.float32),
  )(p0, p1, norm2, w_out)


def kernel(s_embed, r_embed, e_embed, senders, receivers, edge_contr, norm,
           W_s, b_s, W_r, b_r, W_e, W_out, scale1, scale2):
  del edge_contr
  e = senders.shape[0]
  e_pad = ((e + NUM_WORKERS * CHUNK - 1) // (NUM_WORKERS * CHUNK)) * (NUM_WORKERS * CHUNK)

  inv_sqrt2 = np.float32(1.0 / np.sqrt(2.0))
  ws = W_s * inv_sqrt2
  bs = (b_s * inv_sqrt2).reshape(1, MSG)
  wr = W_r * inv_sqrt2
  br = (b_r * inv_sqrt2).reshape(1, MSG)
  we = W_e * (GAIN * scale1)

  s_tab, r_tab = _node_mm(s_embed, ws, bs, r_embed, wr, br)

  ee = jnp.pad(e_embed, ((0, e_pad - e), (0, 0)))
  ew = _ew_mm(ee, we)

  send_pad = jnp.pad(senders, (0, e_pad - e))
  recv_pad = jnp.pad(receivers, (0, e_pad - e))
  zeros = jnp.zeros((N_PAD, MSG), jnp.float32)

  p0, p1 = _make_sc_edge(e_pad)(s_tab, r_tab, ew, send_pad, recv_pad, zeros)

  norm2 = (norm * scale2).reshape(N, 1)
  return _out_mm(p0[:N], p1[:N], norm2, W_out)
